# baseline (speedup 1.0000x reference)
import jax
import jax.numpy as jnp
from jax.experimental import pallas as pl
from jax.experimental.pallas import tpu as pltpu


def _copy_body(u_ref, i_ref, uo_ref, io_ref, u_sem, i_sem):
    u_copy = pltpu.make_async_copy(u_ref, uo_ref, u_sem)
    i_copy = pltpu.make_async_copy(i_ref, io_ref, i_sem)
    u_copy.start()
    i_copy.start()
    u_copy.wait()
    i_copy.wait()


def kernel(user_embeds, item_embeds, adj):
    u_out, i_out = pl.pallas_call(
        _copy_body,
        in_specs=[
            pl.BlockSpec(memory_space=pl.ANY),
            pl.BlockSpec(memory_space=pl.ANY),
        ],
        out_specs=[
            pl.BlockSpec(memory_space=pl.ANY),
            pl.BlockSpec(memory_space=pl.ANY),
        ],
        out_shape=[
            jax.ShapeDtypeStruct(user_embeds.shape, user_embeds.dtype),
            jax.ShapeDtypeStruct(item_embeds.shape, item_embeds.dtype),
        ],
        scratch_shapes=[pltpu.SemaphoreType.DMA, pltpu.SemaphoreType.DMA],
    )(user_embeds, item_embeds)
    return (u_out, i_out)


# speedup vs baseline: 11.8282x; 11.8282x over previous
import jax
import jax.numpy as jnp
from jax.experimental import pallas as pl
from jax.experimental.pallas import tpu as pltpu

_GRID = 125
_WIDE_U = 512
_WIDE_I = 128


def _copy_body(u_ref, i_ref, uo_ref, io_ref):
    uo_ref[...] = u_ref[...]
    io_ref[...] = i_ref[...]


def kernel(user_embeds, item_embeds, adj):
    n_u, d = user_embeds.shape
    n_i, _ = item_embeds.shape
    u_wide = user_embeds.reshape(n_u * d // _WIDE_U, _WIDE_U)
    i_wide = item_embeds.reshape(n_i * d // _WIDE_I, _WIDE_I)
    ub = u_wide.shape[0] // _GRID
    ib = i_wide.shape[0] // _GRID

    u_out, i_out = pl.pallas_call(
        _copy_body,
        grid=(_GRID,),
        in_specs=[
            pl.BlockSpec((ub, _WIDE_U), lambda g: (g, 0)),
            pl.BlockSpec((ib, _WIDE_I), lambda g: (g, 0)),
        ],
        out_specs=[
            pl.BlockSpec((ub, _WIDE_U), lambda g: (g, 0)),
            pl.BlockSpec((ib, _WIDE_I), lambda g: (g, 0)),
        ],
        out_shape=[
            jax.ShapeDtypeStruct(u_wide.shape, u_wide.dtype),
            jax.ShapeDtypeStruct(i_wide.shape, i_wide.dtype),
        ],
    )(u_wide, i_wide)
    return (u_out.reshape(n_u, d), i_out.reshape(n_i, d))


# speedup vs baseline: 16.1150x; 1.3624x over previous
import jax
import jax.numpy as jnp
from jax.experimental import pallas as pl
from jax.experimental.pallas import tpu as pltpu

_GRID = 125


def _copy_body(u_ref, i_ref, uo_ref, io_ref):
    uo_ref[...] = u_ref[...]
    io_ref[...] = i_ref[...]


def kernel(user_embeds, item_embeds, adj):
    n_u, d = user_embeds.shape
    n_i, _ = item_embeds.shape
    ub = n_u // _GRID
    ib = n_i // _GRID

    u_out, i_out = pl.pallas_call(
        _copy_body,
        grid=(_GRID,),
        in_specs=[
            pl.BlockSpec((ub, d), lambda g: (g, 0)),
            pl.BlockSpec((ib, d), lambda g: (g, 0)),
        ],
        out_specs=[
            pl.BlockSpec((ub, d), lambda g: (g, 0)),
            pl.BlockSpec((ib, d), lambda g: (g, 0)),
        ],
        out_shape=[
            jax.ShapeDtypeStruct(user_embeds.shape, user_embeds.dtype),
            jax.ShapeDtypeStruct(item_embeds.shape, item_embeds.dtype),
        ],
        compiler_params=pltpu.CompilerParams(
            dimension_semantics=(pltpu.GridDimensionSemantics.PARALLEL,),
        ),
    )(user_embeds, item_embeds)
    return (u_out, i_out)
